# baseline (speedup 1.0000x reference)
import functools

import jax
import jax.numpy as jnp
from jax import lax
from jax.experimental import pallas as pl
from jax.experimental.pallas import tpu as pltpu
from jax.experimental.pallas import tpu_sc as plsc

B = 16384
D = 32
L = 16
W = 128
NC, NS = 2, 16
NW = NC * NS
BPW = B // NW
NG = BPW // L
NBUF = 14
AHEAD = NBUF - 1


def _body(uid_hbm, iid_hbm, uembt_hbm, iembt_hbm, bias_hbm, out_hbm,
          idv_u, idv_i, buf_u, buf_i, bias_v, acc_v, sem_u, sem_i):
    c = lax.axis_index("c")
    s = lax.axis_index("s")
    wid = s * NC + c
    base = wid * BPW

    pltpu.sync_copy(uid_hbm.at[pl.ds(base, BPW)], idv_u)
    pltpu.sync_copy(iid_hbm.at[pl.ds(base, BPW)], idv_i)
    pltpu.sync_copy(bias_hbm, bias_v)

    def fire(ku, ki, slot):
        offu = pl.multiple_of((ku // W) * W, W)
        offi = pl.multiple_of((ki // W) * W, W)
        pltpu.async_copy(
            uembt_hbm.at[pl.ds(0, D), pl.ds(offu, W)], buf_u.at[slot], sem_u)
        pltpu.async_copy(
            iembt_hbm.at[pl.ds(0, D), pl.ds(offi, W)], buf_i.at[slot], sem_i)

    def drain(slot):
        pltpu.make_async_copy(
            uembt_hbm.at[pl.ds(0, D), pl.ds(0, W)], buf_u.at[slot],
            sem_u).wait()
        pltpu.make_async_copy(
            iembt_hbm.at[pl.ds(0, D), pl.ds(0, W)], buf_i.at[slot],
            sem_i).wait()

    bias = bias_v[...]
    rows_lo = jnp.arange(L, dtype=jnp.int32)
    rows_hi = rows_lo + L

    iu0 = idv_u[pl.ds(0, L)]
    ii0 = idv_i[pl.ds(0, L)]
    for j in range(AHEAD):
        fire(iu0[j], ii0[j], j)

    def group(g, carry):
        iu = idv_u[pl.ds(g * L, L)]
        ii = idv_i[pl.ds(g * L, L)]
        y_vec = jnp.zeros((L,), jnp.float32)
        for j in range(L):
            slot = lax.rem(g * L + j, NBUF)
            nslot = lax.rem(g * L + j + AHEAD, NBUF)
            nj = j + AHEAD
            if nj < L:
                fire(iu[nj], ii[nj], nslot)
            else:
                @pl.when(g + 1 < NG)
                def _():
                    iun = idv_u[pl.ds((g + 1) * L, L)]
                    iin = idv_i[pl.ds((g + 1) * L, L)]
                    fire(iun[nj - L], iin[nj - L], nslot)
            drain(slot)
            cu = jnp.full((L,), lax.rem(iu[j], W), dtype=jnp.int32)
            ci = jnp.full((L,), lax.rem(ii[j], W), dtype=jnp.int32)
            au = plsc.load_gather(buf_u.at[slot], [rows_lo, cu])
            bu = plsc.load_gather(buf_u.at[slot], [rows_hi, cu])
            av = plsc.load_gather(buf_i.at[slot], [rows_lo, ci])
            bv = plsc.load_gather(buf_i.at[slot], [rows_hi, ci])
            dot = lax.reduce_sum(au * av + bu * bv, axes=(0,))
            y_vec = jnp.where(rows_lo == j, dot, y_vec)
        acc_v[pl.ds(g * L, L)] = 1.0 / (1.0 + jnp.exp(-(y_vec + bias)))
        return carry

    lax.fori_loop(0, NG, group, 0)

    pltpu.sync_copy(acc_v, out_hbm.at[pl.ds(base, BPW)])


def kernel(user_id, item_id, user_emb, item_emb, bias):
    uid = user_id.astype(jnp.int32)
    iid = item_id.astype(jnp.int32)
    uembt = user_emb.T
    iembt = item_emb.T
    bias_vec = jnp.full((L,), bias, jnp.float32)
    mesh = plsc.VectorSubcoreMesh(core_axis_name="c", subcore_axis_name="s")
    k = functools.partial(
        pl.kernel,
        mesh=mesh,
        compiler_params=pltpu.CompilerParams(
            needs_layout_passes=False, disable_bounds_checks=True),
        out_type=jax.ShapeDtypeStruct((B,), jnp.float32),
        scratch_types=[
            pltpu.VMEM((BPW,), jnp.int32),
            pltpu.VMEM((BPW,), jnp.int32),
            pltpu.VMEM((NBUF, D, W), jnp.float32),
            pltpu.VMEM((NBUF, D, W), jnp.float32),
            pltpu.VMEM((L,), jnp.float32),
            pltpu.VMEM((BPW,), jnp.float32),
            pltpu.SemaphoreType.DMA,
            pltpu.SemaphoreType.DMA,
        ],
    )(_body)
    out = k(uid, iid, uembt, iembt, bias_vec)
    return jnp.reshape(out, (B, 1))


# speedup vs baseline: 1.0077x; 1.0077x over previous
import functools

import jax
import jax.numpy as jnp
from jax import lax
from jax.experimental import pallas as pl
from jax.experimental.pallas import tpu as pltpu
from jax.experimental.pallas import tpu_sc as plsc

B = 16384
D = 32
L = 16
W = 128
NC, NS = 2, 16
NW = NC * NS
BPW = B // NW
NG = BPW // L
NBUF = 8
AHEAD = NBUF - 1


def _body(uid_hbm, iid_hbm, uembt_hbm, iembt_hbm, bias_hbm, out_hbm,
          idv_u, idv_i, buf_u, buf_i, bias_v, acc_v, sem_u, sem_i):
    c = lax.axis_index("c")
    s = lax.axis_index("s")
    wid = s * NC + c
    base = wid * BPW

    pltpu.sync_copy(uid_hbm.at[pl.ds(base, BPW)], idv_u)
    pltpu.sync_copy(iid_hbm.at[pl.ds(base, BPW)], idv_i)
    pltpu.sync_copy(bias_hbm, bias_v)

    def fire(ku, ki, slot):
        offu = pl.multiple_of((ku // W) * W, W)
        offi = pl.multiple_of((ki // W) * W, W)
        pltpu.async_copy(
            uembt_hbm.at[pl.ds(0, D), pl.ds(offu, W)], buf_u.at[slot], sem_u)
        pltpu.async_copy(
            iembt_hbm.at[pl.ds(0, D), pl.ds(offi, W)], buf_i.at[slot], sem_i)

    def drain(slot):
        pltpu.make_async_copy(
            uembt_hbm.at[pl.ds(0, D), pl.ds(0, W)], buf_u.at[slot],
            sem_u).wait()
        pltpu.make_async_copy(
            iembt_hbm.at[pl.ds(0, D), pl.ds(0, W)], buf_i.at[slot],
            sem_i).wait()

    bias = bias_v[...]
    rows_lo = jnp.arange(L, dtype=jnp.int32)
    rows_hi = rows_lo + L

    iu0 = idv_u[pl.ds(0, L)]
    ii0 = idv_i[pl.ds(0, L)]
    for j in range(AHEAD):
        fire(iu0[j], ii0[j], j)

    def group(g, carry):
        iu = idv_u[pl.ds(g * L, L)]
        ii = idv_i[pl.ds(g * L, L)]
        y_vec = jnp.zeros((L,), jnp.float32)
        for j in range(L):
            slot = j % NBUF
            nslot = (j + AHEAD) % NBUF
            nj = j + AHEAD
            if nj < L:
                fire(iu[nj], ii[nj], nslot)
            else:
                @pl.when(g + 1 < NG)
                def _():
                    iun = idv_u[pl.ds((g + 1) * L, L)]
                    iin = idv_i[pl.ds((g + 1) * L, L)]
                    fire(iun[nj - L], iin[nj - L], nslot)
            drain(slot)
            cu = jnp.full((L,), lax.rem(iu[j], W), dtype=jnp.int32)
            ci = jnp.full((L,), lax.rem(ii[j], W), dtype=jnp.int32)
            au = plsc.load_gather(buf_u.at[slot], [rows_lo, cu])
            bu = plsc.load_gather(buf_u.at[slot], [rows_hi, cu])
            av = plsc.load_gather(buf_i.at[slot], [rows_lo, ci])
            bv = plsc.load_gather(buf_i.at[slot], [rows_hi, ci])
            dot = lax.reduce_sum(au * av + bu * bv, axes=(0,))
            y_vec = jnp.where(rows_lo == j, dot, y_vec)
        acc_v[pl.ds(g * L, L)] = 1.0 / (1.0 + jnp.exp(-(y_vec + bias)))
        return carry

    lax.fori_loop(0, NG, group, 0)

    pltpu.sync_copy(acc_v, out_hbm.at[pl.ds(base, BPW)])


def kernel(user_id, item_id, user_emb, item_emb, bias):
    uid = user_id.astype(jnp.int32)
    iid = item_id.astype(jnp.int32)
    uembt = user_emb.T
    iembt = item_emb.T
    bias_vec = jnp.full((L,), bias, jnp.float32)
    mesh = plsc.VectorSubcoreMesh(core_axis_name="c", subcore_axis_name="s")
    k = functools.partial(
        pl.kernel,
        mesh=mesh,
        compiler_params=pltpu.CompilerParams(
            needs_layout_passes=False, disable_bounds_checks=True),
        out_type=jax.ShapeDtypeStruct((B,), jnp.float32),
        scratch_types=[
            pltpu.VMEM((BPW,), jnp.int32),
            pltpu.VMEM((BPW,), jnp.int32),
            pltpu.VMEM((NBUF, D, W), jnp.float32),
            pltpu.VMEM((NBUF, D, W), jnp.float32),
            pltpu.VMEM((L,), jnp.float32),
            pltpu.VMEM((BPW,), jnp.float32),
            pltpu.SemaphoreType.DMA,
            pltpu.SemaphoreType.DMA,
        ],
    )(_body)
    out = k(uid, iid, uembt, iembt, bias_vec)
    return jnp.reshape(out, (B, 1))
